# Initial kernel scaffold; baseline (speedup 1.0000x reference)
import jax
import jax.numpy as jnp
from jax.experimental import pallas as pl


def kernel(z, x, idx_i, idx_j, d_ij):
    raise NotImplementedError("write your pallas kernel here")


# speedup vs baseline: 2.9520x; 2.9520x over previous
import functools

import jax
import jax.numpy as jnp
import numpy as np
from jax import lax
from jax.experimental import pallas as pl
from jax.experimental.pallas import tpu as pltpu
from jax.experimental.pallas import tpu_sc as plsc

_ORBITALS = '1s 2s 2p 3s 3p 4s 3d 4p 5s 4d 5p 6s 4f 5d 6p 7s 5f 6d 7p 6f 7d 7f'.split()
_POSSIBLE_ELECTRONS = dict(s=2, p=6, d=10, f=14)


def _electron_config(atomic_num):
    electron_count, last_idx, config = 0, -1, []
    for i in _ORBITALS:
        if electron_count < atomic_num:
            config.append(_POSSIBLE_ELECTRONS[i[-1]])
            electron_count += _POSSIBLE_ELECTRONS[i[-1]]
            last_idx += 1
        else:
            config.append(0)
    if electron_count > atomic_num:
        config[last_idx] -= electron_count - atomic_num
    return config


_CFG = np.asarray([_electron_config(a) for a in range(100)], dtype=np.float32)
_PAIR = np.zeros((100 * 100, 128), dtype=np.float32)
_PAIR[:, :22] = np.repeat(_CFG, 100, axis=0)
_PAIR[:, 22:44] = np.tile(_CFG, (100, 1))

_D = 256
_A = 22
_OUT_W = 2 * _D + 2 * _A + 1


@functools.partial(jax.jit, static_argnums=(5, 6))
def _run(z2, x, idx_i, idx_j, d_ij, E, B):
    NW = 32
    per_w = E // NW
    nch = per_w // B

    mesh = plsc.VectorSubcoreMesh(core_axis_name="c", subcore_axis_name="s")

    Bp = ((B + 15) // 16) * 16
    NG = Bp // 16

    @functools.partial(
        pl.kernel,
        mesh=mesh,
        compiler_params=pltpu.CompilerParams(needs_layout_passes=False),
        out_type=jax.ShapeDtypeStruct((E, _OUT_W), jnp.float32),
        scratch_types=[
            pltpu.VMEM((B,), jnp.int32),
            pltpu.VMEM((B,), jnp.int32),
            pltpu.VMEM((Bp,), jnp.int32),
            pltpu.VMEM((Bp,), jnp.int32),
            pltpu.VMEM((Bp,), jnp.int32),
            pltpu.VMEM((Bp,), jnp.float32),
            pltpu.VMEM((B, 128), jnp.float32),
            pltpu.VMEM((B, _OUT_W), jnp.float32),
            pltpu.SemaphoreType.DMA,
            pltpu.SemaphoreType.DMA,
            pltpu.SemaphoreType.DMA,
            pltpu.SemaphoreType.DMA,
            pltpu.SemaphoreType.DMA,
            pltpu.SemaphoreType.DMA,
        ],
    )
    def k(z_hbm, x_hbm, pair_hbm, ii_hbm, jj_hbm, d_hbm, out_hbm,
          ii_v, jj_v, zi_v, zj_v, ci_v, d_v, ab_v, rows_v,
          sem_xi, sem_xj, sem_zi, sem_zj, sem_d, sem_ab):
        wid = lax.axis_index("s") * 2 + lax.axis_index("c")
        w_base = wid * per_w

        def body(c, carry):
            base = w_base + c * B
            pltpu.sync_copy(ii_hbm.at[pl.ds(base, B)], ii_v)
            pltpu.sync_copy(jj_hbm.at[pl.ds(base, B)], jj_v)
            cp_xi = pltpu.async_copy(
                x_hbm.at[ii_v], rows_v.at[pl.ds(0, B), pl.ds(0, _D)], sem_xi)
            cp_xj = pltpu.async_copy(
                x_hbm.at[jj_v], rows_v.at[pl.ds(0, B), pl.ds(_D, _D)], sem_xj)
            cp_zi = pltpu.async_copy(
                z_hbm.at[ii_v], zi_v.at[pl.ds(0, B)], sem_zi)
            cp_zj = pltpu.async_copy(
                z_hbm.at[jj_v], zj_v.at[pl.ds(0, B)], sem_zj)
            cp_d = pltpu.async_copy(
                d_hbm.at[pl.ds(base, B)], d_v.at[pl.ds(0, B)], sem_d)
            cp_zi.wait()
            cp_zj.wait()
            for t in range(NG):
                sl = pl.ds(t * 16, 16)
                ci_v[sl] = zi_v[sl] * 100 + zj_v[sl]
            cp_ab = pltpu.async_copy(
                pair_hbm.at[ci_v.at[pl.ds(0, B)]], ab_v, sem_ab)
            cp_ab.wait()
            cp_d.wait()
            lane = lax.iota(jnp.int32, 16)
            col556 = jnp.full((16,), 2 * _D + 2 * _A, jnp.int32)
            for t in range(NG):
                rows16 = lane + t * 16
                plsc.store_scatter(rows_v, [rows16, col556],
                                   d_v[pl.ds(t * 16, 16)], mask=rows16 < B)
            tail_cols = lane + (2 * _D + 2 * _A - 12)
            tail_msk = lane < 12
            for r in range(B):
                rows_v[r, pl.ds(2 * _D, 16)] = ab_v[r, pl.ds(0, 16)]
                rows_v[r, pl.ds(2 * _D + 16, 16)] = ab_v[r, pl.ds(16, 16)]
                plsc.store_scatter(
                    rows_v, [jnp.full((16,), r, jnp.int32), tail_cols],
                    ab_v[r, pl.ds(32, 16)], mask=tail_msk)
            cp_xi.wait()
            cp_xj.wait()
            pltpu.sync_copy(rows_v, out_hbm.at[pl.ds(base, B)])
            return carry

        lax.fori_loop(0, nch, body, 0)

    return k(z2, x, jnp.asarray(_PAIR), idx_i, idx_j, d_ij)


def kernel(z, x, idx_i, idx_j, d_ij):
    E = idx_i.shape[0]
    z2 = z.astype(jnp.int32)
    idx_i = idx_i.astype(jnp.int32)
    idx_j = idx_j.astype(jnp.int32)
    return _run(z2, x, idx_i, idx_j, d_ij.astype(jnp.float32), E, 40)


# speedup vs baseline: 3.8890x; 1.3174x over previous
import functools

import jax
import jax.numpy as jnp
import numpy as np
from jax import lax
from jax.experimental import pallas as pl
from jax.experimental.pallas import tpu as pltpu
from jax.experimental.pallas import tpu_sc as plsc

_ORBITALS = '1s 2s 2p 3s 3p 4s 3d 4p 5s 4d 5p 6s 4f 5d 6p 7s 5f 6d 7p 6f 7d 7f'.split()
_POSSIBLE_ELECTRONS = dict(s=2, p=6, d=10, f=14)


def _electron_config(atomic_num):
    electron_count, last_idx, config = 0, -1, []
    for i in _ORBITALS:
        if electron_count < atomic_num:
            config.append(_POSSIBLE_ELECTRONS[i[-1]])
            electron_count += _POSSIBLE_ELECTRONS[i[-1]]
            last_idx += 1
        else:
            config.append(0)
    if electron_count > atomic_num:
        config[last_idx] -= electron_count - atomic_num
    return config


_CFG = np.asarray([_electron_config(a) for a in range(100)], dtype=np.float32)
_PAIR = np.zeros((100 * 100, 128), dtype=np.float32)
_PAIR[:, :22] = np.repeat(_CFG, 100, axis=0)
_PAIR[:, 22:44] = np.tile(_CFG, (100, 1))

_D = 256
_A = 22
_OUT_W = 2 * _D + 2 * _A + 1


@functools.partial(jax.jit, static_argnums=(5, 6))
def _run(z2, x, idx_i, idx_j, d_ij, E, B):
    NW = 32
    per_w = E // NW
    nch = per_w // B
    assert nch % 2 == 1 and nch >= 3
    nch2 = (nch - 1) // 2

    Bp = ((B + 15) // 16) * 16
    NG = Bp // 16

    mesh = plsc.VectorSubcoreMesh(core_axis_name="c", subcore_axis_name="s")

    set_types = [
        pltpu.VMEM((B,), jnp.int32),
        pltpu.VMEM((B,), jnp.int32),
        pltpu.VMEM((Bp,), jnp.int32),
        pltpu.VMEM((Bp,), jnp.int32),
        pltpu.VMEM((Bp,), jnp.int32),
        pltpu.VMEM((Bp,), jnp.float32),
        pltpu.VMEM((B, 128), jnp.float32),
        pltpu.VMEM((B, _OUT_W), jnp.float32),
    ] + [pltpu.SemaphoreType.DMA] * 9

    @functools.partial(
        pl.kernel,
        mesh=mesh,
        compiler_params=pltpu.CompilerParams(needs_layout_passes=False),
        out_type=jax.ShapeDtypeStruct((E, _OUT_W), jnp.float32),
        scratch_types=set_types + set_types,
    )
    def k(z_hbm, x_hbm, pair_hbm, ii_hbm, jj_hbm, d_hbm, out_hbm, *sc):
        half = len(sc) // 2
        S = [sc[:half], sc[half:]]
        wid = lax.axis_index("s") * 2 + lax.axis_index("c")
        w_base = wid * per_w

        lane = lax.iota(jnp.int32, 16)
        col556 = jnp.full((16,), 2 * _D + 2 * _A, jnp.int32)
        tail_cols = lane + (2 * _D + 2 * _A - 12)
        tail_msk = lane < 12

        def idx_issue(p, chunk):
            (ii_v, jj_v, _, _, _, d_v, _, _,
             s_ii, s_jj, _, _, _, _, s_d, _, _) = S[p]
            base = w_base + chunk * B
            pltpu.async_copy(ii_hbm.at[pl.ds(base, B)], ii_v, s_ii)
            pltpu.async_copy(jj_hbm.at[pl.ds(base, B)], jj_v, s_jj)
            pltpu.async_copy(d_hbm.at[pl.ds(base, B)], d_v.at[pl.ds(0, B)],
                             s_d)

        def front(p):
            (ii_v, jj_v, zi_v, zj_v, ci_v, _, ab_v, rows_v,
             s_ii, s_jj, s_xi, s_xj, s_zi, s_zj, _, s_ab, _) = S[p]
            pltpu.make_async_copy(
                ii_hbm.at[pl.ds(0, B)], ii_v, s_ii).wait()
            pltpu.make_async_copy(
                jj_hbm.at[pl.ds(0, B)], jj_v, s_jj).wait()
            pltpu.async_copy(
                x_hbm.at[ii_v], rows_v.at[pl.ds(0, B), pl.ds(0, _D)], s_xi)
            pltpu.async_copy(
                x_hbm.at[jj_v], rows_v.at[pl.ds(0, B), pl.ds(_D, _D)], s_xj)
            pltpu.async_copy(z_hbm.at[ii_v], zi_v.at[pl.ds(0, B)], s_zi)
            pltpu.async_copy(z_hbm.at[jj_v], zj_v.at[pl.ds(0, B)], s_zj)
            pltpu.make_async_copy(
                z_hbm.at[ii_v], zi_v.at[pl.ds(0, B)], s_zi).wait()
            pltpu.make_async_copy(
                z_hbm.at[jj_v], zj_v.at[pl.ds(0, B)], s_zj).wait()
            for t in range(NG):
                sl = pl.ds(t * 16, 16)
                ci_v[sl] = zi_v[sl] * 100 + zj_v[sl]
            pltpu.async_copy(pair_hbm.at[ci_v.at[pl.ds(0, B)]], ab_v, s_ab)

        def back(p, chunk):
            (ii_v, jj_v, _, _, ci_v, d_v, ab_v, rows_v,
             _, _, s_xi, s_xj, _, _, s_d, s_ab, s_wr) = S[p]
            base = w_base + chunk * B
            pltpu.make_async_copy(
                pair_hbm.at[ci_v.at[pl.ds(0, B)]], ab_v, s_ab).wait()
            pltpu.make_async_copy(
                d_hbm.at[pl.ds(0, B)], d_v.at[pl.ds(0, B)], s_d).wait()
            for t in range(NG):
                rows16 = lane + t * 16
                plsc.store_scatter(rows_v, [rows16, col556],
                                   d_v[pl.ds(t * 16, 16)], mask=rows16 < B)
            for r in range(B):
                rows_v[r, pl.ds(2 * _D, 16)] = ab_v[r, pl.ds(0, 16)]
                rows_v[r, pl.ds(2 * _D + 16, 16)] = ab_v[r, pl.ds(16, 16)]
                plsc.store_scatter(
                    rows_v, [jnp.full((16,), r, jnp.int32), tail_cols],
                    ab_v[r, pl.ds(32, 16)], mask=tail_msk)
            pltpu.make_async_copy(
                x_hbm.at[ii_v], rows_v.at[pl.ds(0, B), pl.ds(0, _D)],
                s_xi).wait()
            pltpu.make_async_copy(
                x_hbm.at[jj_v], rows_v.at[pl.ds(0, B), pl.ds(_D, _D)],
                s_xj).wait()
            pltpu.async_copy(rows_v, out_hbm.at[pl.ds(base, B)], s_wr)

        def wr_wait(p):
            rows_v, s_wr = S[p][7], S[p][16]
            pltpu.make_async_copy(
                rows_v, out_hbm.at[pl.ds(0, B)], s_wr).wait()

        idx_issue(0, 0)
        idx_issue(1, 1)

        def body(t, carry):
            a = 2 * t

            @pl.when(t > 0)
            def _():
                wr_wait(0)

            front(0)

            @pl.when(t > 0)
            def _():
                wr_wait(1)

            front(1)
            back(0, a)
            idx_issue(0, a + 2)
            back(1, a + 1)

            @pl.when(t < nch2 - 1)
            def _():
                idx_issue(1, a + 3)

            return carry

        lax.fori_loop(0, nch2, body, 0)

        wr_wait(0)
        front(0)
        back(0, nch - 1)
        wr_wait(0)
        wr_wait(1)

    return k(z2, x, jnp.asarray(_PAIR), idx_i, idx_j, d_ij)


def kernel(z, x, idx_i, idx_j, d_ij):
    E = idx_i.shape[0]
    z2 = z.astype(jnp.int32)
    idx_i = idx_i.astype(jnp.int32)
    idx_j = idx_j.astype(jnp.int32)
    return _run(z2, x, idx_i, idx_j, d_ij.astype(jnp.float32), E, 40)


# speedup vs baseline: 3.9714x; 1.0212x over previous
import functools

import jax
import jax.numpy as jnp
import numpy as np
from jax import lax
from jax.experimental import pallas as pl
from jax.experimental.pallas import tpu as pltpu
from jax.experimental.pallas import tpu_sc as plsc

_ORBITALS = '1s 2s 2p 3s 3p 4s 3d 4p 5s 4d 5p 6s 4f 5d 6p 7s 5f 6d 7p 6f 7d 7f'.split()
_POSSIBLE_ELECTRONS = dict(s=2, p=6, d=10, f=14)


def _electron_config(atomic_num):
    electron_count, last_idx, config = 0, -1, []
    for i in _ORBITALS:
        if electron_count < atomic_num:
            config.append(_POSSIBLE_ELECTRONS[i[-1]])
            electron_count += _POSSIBLE_ELECTRONS[i[-1]]
            last_idx += 1
        else:
            config.append(0)
    if electron_count > atomic_num:
        config[last_idx] -= electron_count - atomic_num
    return config


_CFG = np.asarray([_electron_config(a) for a in range(100)], dtype=np.float32)
_PAIR = np.zeros((100 * 100, 128), dtype=np.float32)
_PAIR[:, :22] = np.repeat(_CFG, 100, axis=0)
_PAIR[:, 22:44] = np.tile(_CFG, (100, 1))

_D = 256
_A = 22
_OUT_W = 2 * _D + 2 * _A + 1


@functools.partial(jax.jit, static_argnums=(5, 6))
def _run(z2, x, idx_i, idx_j, d_ij, E, B):
    NW = 32
    nch_total = E // B
    nfull = nch_total // NW
    nrem = nch_total % NW
    assert nfull % 2 == 0 and nfull >= 4

    Bp = ((B + 15) // 16) * 16
    NG = Bp // 16

    mesh = plsc.VectorSubcoreMesh(core_axis_name="c", subcore_axis_name="s")

    set_types = [
        pltpu.VMEM((B,), jnp.int32),
        pltpu.VMEM((B,), jnp.int32),
        pltpu.VMEM((Bp,), jnp.int32),
        pltpu.VMEM((Bp,), jnp.int32),
        pltpu.VMEM((Bp,), jnp.int32),
        pltpu.VMEM((Bp,), jnp.float32),
        pltpu.VMEM((B, 128), jnp.float32),
        pltpu.VMEM((B, _OUT_W), jnp.float32),
    ] + [pltpu.SemaphoreType.DMA] * 9

    @functools.partial(
        pl.kernel,
        mesh=mesh,
        compiler_params=pltpu.CompilerParams(needs_layout_passes=False),
        out_type=jax.ShapeDtypeStruct((E, _OUT_W), jnp.float32),
        scratch_types=set_types + set_types,
    )
    def k(z_hbm, x_hbm, pair_hbm, ii_hbm, jj_hbm, d_hbm, out_hbm, *sc):
        half = len(sc) // 2
        S = [sc[:half], sc[half:]]
        wid = lax.axis_index("s") * 2 + lax.axis_index("c")
        n_w = nfull + jnp.where(wid < nrem, 1, 0)

        lane = lax.iota(jnp.int32, 16)
        col556 = jnp.full((16,), 2 * _D + 2 * _A, jnp.int32)
        tail_cols = lane + (2 * _D + 2 * _A - 12)
        tail_msk = lane < 12

        def idx_issue(p, chunk):
            (ii_v, jj_v, _, _, _, d_v, _, _,
             s_ii, s_jj, _, _, _, _, s_d, _, _) = S[p]
            base = (wid + NW * chunk) * B
            pltpu.async_copy(ii_hbm.at[pl.ds(base, B)], ii_v, s_ii)
            pltpu.async_copy(jj_hbm.at[pl.ds(base, B)], jj_v, s_jj)
            pltpu.async_copy(d_hbm.at[pl.ds(base, B)], d_v.at[pl.ds(0, B)],
                             s_d)

        def front(p):
            (ii_v, jj_v, zi_v, zj_v, ci_v, _, ab_v, rows_v,
             s_ii, s_jj, s_xi, s_xj, s_zi, s_zj, _, s_ab, _) = S[p]
            pltpu.make_async_copy(
                ii_hbm.at[pl.ds(0, B)], ii_v, s_ii).wait()
            pltpu.make_async_copy(
                jj_hbm.at[pl.ds(0, B)], jj_v, s_jj).wait()
            pltpu.async_copy(
                x_hbm.at[ii_v], rows_v.at[pl.ds(0, B), pl.ds(0, _D)], s_xi)
            pltpu.async_copy(
                x_hbm.at[jj_v], rows_v.at[pl.ds(0, B), pl.ds(_D, _D)], s_xj)
            pltpu.async_copy(z_hbm.at[ii_v], zi_v.at[pl.ds(0, B)], s_zi)
            pltpu.async_copy(z_hbm.at[jj_v], zj_v.at[pl.ds(0, B)], s_zj)
            pltpu.make_async_copy(
                z_hbm.at[ii_v], zi_v.at[pl.ds(0, B)], s_zi).wait()
            pltpu.make_async_copy(
                z_hbm.at[jj_v], zj_v.at[pl.ds(0, B)], s_zj).wait()
            for t in range(NG):
                sl = pl.ds(t * 16, 16)
                ci_v[sl] = zi_v[sl] * 100 + zj_v[sl]
            pltpu.async_copy(pair_hbm.at[ci_v.at[pl.ds(0, B)]], ab_v, s_ab)

        def back(p, chunk):
            (ii_v, jj_v, _, _, ci_v, d_v, ab_v, rows_v,
             _, _, s_xi, s_xj, _, _, s_d, s_ab, s_wr) = S[p]
            base = (wid + NW * chunk) * B
            pltpu.make_async_copy(
                pair_hbm.at[ci_v.at[pl.ds(0, B)]], ab_v, s_ab).wait()
            pltpu.make_async_copy(
                d_hbm.at[pl.ds(0, B)], d_v.at[pl.ds(0, B)], s_d).wait()
            for t in range(NG):
                rows16 = lane + t * 16
                plsc.store_scatter(rows_v, [rows16, col556],
                                   d_v[pl.ds(t * 16, 16)], mask=rows16 < B)
            for r in range(B):
                rows_v[r, pl.ds(2 * _D, 16)] = ab_v[r, pl.ds(0, 16)]
                rows_v[r, pl.ds(2 * _D + 16, 16)] = ab_v[r, pl.ds(16, 16)]
                plsc.store_scatter(
                    rows_v, [jnp.full((16,), r, jnp.int32), tail_cols],
                    ab_v[r, pl.ds(32, 16)], mask=tail_msk)
            pltpu.make_async_copy(
                x_hbm.at[ii_v], rows_v.at[pl.ds(0, B), pl.ds(0, _D)],
                s_xi).wait()
            pltpu.make_async_copy(
                x_hbm.at[jj_v], rows_v.at[pl.ds(0, B), pl.ds(_D, _D)],
                s_xj).wait()
            pltpu.async_copy(rows_v, out_hbm.at[pl.ds(base, B)], s_wr)

        def wr_wait(p):
            rows_v, s_wr = S[p][7], S[p][16]
            pltpu.make_async_copy(
                rows_v, out_hbm.at[pl.ds(0, B)], s_wr).wait()

        idx_issue(0, 0)
        idx_issue(1, 1)

        def body(t, carry):
            a = 2 * t

            @pl.when(t > 0)
            def _():
                wr_wait(0)

            front(0)

            @pl.when(t > 0)
            def _():
                wr_wait(1)

            front(1)
            back(0, a)

            @pl.when(a + 2 < n_w)
            def _():
                idx_issue(0, a + 2)

            back(1, a + 1)

            @pl.when(a + 3 < n_w)
            def _():
                idx_issue(1, a + 3)

            return carry

        lax.fori_loop(0, nfull // 2, body, 0)

        @pl.when(wid < nrem)
        def _():
            wr_wait(0)
            front(0)
            back(0, nfull)

        wr_wait(0)
        wr_wait(1)

    return k(z2, x, jnp.asarray(_PAIR), idx_i, idx_j, d_ij)


def kernel(z, x, idx_i, idx_j, d_ij):
    E = idx_i.shape[0]
    z2 = z.astype(jnp.int32)
    idx_i = idx_i.astype(jnp.int32)
    idx_j = idx_j.astype(jnp.int32)
    return _run(z2, x, idx_i, idx_j, d_ij.astype(jnp.float32), E, 64)
